# baseline (speedup 1.0000x reference)
import functools

import jax
import jax.numpy as jnp
from jax import lax
from jax.experimental import pallas as pl
from jax.experimental.pallas import tpu as pltpu
from jax.experimental.pallas import tpu_sc as plsc

NC = 2
NS = 16
NW = NC * NS
CH = 128


def _gather_kernel(idx_hbm, table_hbm, out_hbm, idx_v, rows_v, sem):
    n_ch = idx_v.shape[0]
    wid = lax.axis_index("s") * NC + lax.axis_index("c")
    pltpu.sync_copy(idx_hbm.at[wid], idx_v)

    def body(j, _):
        pltpu.async_copy(table_hbm.at[idx_v.at[j]], rows_v, sem).wait()
        pltpu.sync_copy(rows_v, out_hbm.at[wid, pl.ds(j * CH, CH)])
        return ()

    lax.fori_loop(0, n_ch, body, (), unroll=False)


def kernel(indices, table):
    B0, B1 = indices.shape
    V, D = table.shape
    B = B0 * B1
    assert B % (NW * CH) == 0
    b_per_w = B // NW
    n_ch = b_per_w // CH

    idx = indices.reshape(NW, n_ch, CH).astype(jnp.int32)

    mesh = plsc.VectorSubcoreMesh(core_axis_name="c", subcore_axis_name="s")
    k = functools.partial(
        pl.kernel,
        mesh=mesh,
        out_type=jax.ShapeDtypeStruct((NW, b_per_w, D), jnp.float32),
        scratch_types=[
            pltpu.VMEM((n_ch, CH), jnp.int32),
            pltpu.VMEM((CH, D), jnp.float32),
            pltpu.SemaphoreType.DMA,
        ],
        compiler_params=pltpu.CompilerParams(use_tc_tiling_on_sc=False),
    )(_gather_kernel)
    out = k(idx, table)
    return out.reshape(B0, B1, D)


# speedup vs baseline: 1.0480x; 1.0480x over previous
import functools

import jax
import jax.numpy as jnp
from jax import lax
from jax.experimental import pallas as pl
from jax.experimental.pallas import tpu as pltpu
from jax.experimental.pallas import tpu_sc as plsc

NC = 2
NS = 16
NW = NC * NS
CH = 128
K = 5


def _gather_kernel(idx_hbm, table_hbm, out_hbm, idx_v, rows_v, gsem, wsem):
    n_ch = idx_v.shape[0]
    n_grp = n_ch // K
    wid = lax.axis_index("s") * NC + lax.axis_index("c")
    pltpu.sync_copy(idx_hbm.at[wid], idx_v)

    def start_g(g, s):
        for b in range(K):
            pltpu.async_copy(
                table_hbm.at[idx_v.at[g * K + b]], rows_v.at[s, b], gsem)

    def wait_g(g, s):
        for b in range(K):
            pltpu.make_async_copy(
                table_hbm.at[idx_v.at[g * K + b]], rows_v.at[s, b], gsem
            ).wait()

    def start_w(g, s):
        for b in range(K):
            pltpu.async_copy(
                rows_v.at[s, b],
                out_hbm.at[wid, pl.ds((g * K + b) * CH, CH)], wsem)

    def wait_w(g, s):
        for b in range(K):
            pltpu.make_async_copy(
                rows_v.at[s, b],
                out_hbm.at[wid, pl.ds((g * K + b) * CH, CH)], wsem
            ).wait()

    start_g(0, 0)

    def pair(p, _):
        for sub in range(2):
            g = 2 * p + sub
            o = 1 - sub

            @pl.when(jnp.logical_and(g >= 1, g + 1 < n_grp))
            def _():
                wait_w(g - 1, o)

            @pl.when(g + 1 < n_grp)
            def _():
                start_g(g + 1, o)

            wait_g(g, sub)
            start_w(g, sub)
        return ()

    lax.fori_loop(0, n_grp // 2, pair, (), unroll=False)
    wait_w(n_grp - 2, 0)
    wait_w(n_grp - 1, 1)


def kernel(indices, table):
    B0, B1 = indices.shape
    V, D = table.shape
    B = B0 * B1
    assert B % (NW * CH) == 0
    b_per_w = B // NW
    n_ch = b_per_w // CH
    assert n_ch % (2 * K) == 0

    idx = indices.reshape(NW, n_ch, CH).astype(jnp.int32)

    mesh = plsc.VectorSubcoreMesh(core_axis_name="c", subcore_axis_name="s")
    k = functools.partial(
        pl.kernel,
        mesh=mesh,
        out_type=jax.ShapeDtypeStruct((NW, b_per_w, D), jnp.float32),
        scratch_types=[
            pltpu.VMEM((n_ch, CH), jnp.int32),
            pltpu.VMEM((2, K, CH, D), jnp.float32),
            pltpu.SemaphoreType.DMA,
            pltpu.SemaphoreType.DMA,
        ],
        compiler_params=pltpu.CompilerParams(use_tc_tiling_on_sc=False),
    )(_gather_kernel)
    out = k(idx, table)
    return out.reshape(B0, B1, D)


# speedup vs baseline: 1.1561x; 1.1031x over previous
import functools

import jax
import jax.numpy as jnp
from jax import lax
from jax.experimental import pallas as pl
from jax.experimental.pallas import tpu as pltpu
from jax.experimental.pallas import tpu_sc as plsc

NC = 2
NS = 16
NW = NC * NS
B0 = 4096
B1 = 50
D = 32
BW = B0 // NW
SG = 5
NSG = B1 // SG
NBG = BW // 16
NCH = NSG * NBG
CHUNK = 16 * SG


def _emb_kernel(idxT_hbm, tab_hbm, out_hbm,
                idxT_v, idxf_v, subT_v, gbuf_v, obuf_v, gsem, wsem):
    wid = lax.axis_index("s") * NC + lax.axis_index("c")
    bbase = wid * BW

    pltpu.sync_copy(idxT_hbm.at[pl.ds(0, B1), pl.ds(bbase, BW)],
                    idxT_v.at[pl.ds(0, B1)])

    iota = lax.iota(jnp.int32, 16)

    def pidx(c, _):
        g = lax.shift_right_logical(c, 3)
        bg = jnp.bitwise_and(c, 7)
        for sl in range(SG):
            s = g * SG + sl
            svec = jnp.full((16,), s, jnp.int32)
            bcol = bg * 16 + iota
            v = plsc.load_gather(idxT_v, [svec, bcol])
            base = c * CHUNK + sl * 16
            idxf_v[pl.ds(base, 16)] = lax.shift_right_logical(v, 2)
            subT_v[pl.ds(base, 16)] = lax.shift_left(
                jnp.bitwise_and(v, 3), 5)
        return ()

    lax.fori_loop(0, NCH, pidx, (), unroll=False)

    def start_g(c, bs):
        pltpu.async_copy(
            tab_hbm.at[idxf_v.at[pl.ds(c * CHUNK, CHUNK)]],
            gbuf_v.at[bs], gsem)

    def wait_g(c, bs):
        pltpu.make_async_copy(
            tab_hbm.at[idxf_v.at[pl.ds(c * CHUNK, CHUNK)]],
            gbuf_v.at[bs], gsem).wait()

    def out_dst(g):
        return out_hbm.at[pl.ds(g * SG, SG), pl.ds(0, D),
                          pl.ds(bbase, BW)]

    def extract(c, bs, gslot, bg):
        for sl in range(SG):
            sub = subT_v[pl.ds(c * CHUNK + sl * 16, 16)]
            rowv = sl * 16 + iota
            slotv = jnp.full((16,), bs, jnp.int32)
            gv = jnp.full((16,), gslot, jnp.int32)
            slv = jnp.full((16,), sl, jnp.int32)
            bv = bg * 16 + iota

            def jbody(j, carry):
                colv, jv = carry
                v = plsc.load_gather(gbuf_v, [slotv, rowv, colv])
                plsc.store_scatter(obuf_v, [gv, slv, jv, bv], v)
                return (colv + 1, jv + 1)

            lax.fori_loop(0, D, jbody,
                          (sub, jnp.zeros((16,), jnp.int32)), unroll=8)

    def start_w(g, gslot):
        pltpu.async_copy(obuf_v.at[gslot], out_dst(g), wsem)

    def wait_w(g, gslot):
        pltpu.make_async_copy(obuf_v.at[gslot], out_dst(g), wsem).wait()

    start_g(0, 0)

    def gpair(p, _):
        for gsub in range(2):
            g = 2 * p + gsub

            @pl.when(g >= 2)
            def _():
                wait_w(g - 2, gsub)

            for bg in range(NBG):
                c = g * NBG + bg

                @pl.when(c + 1 < NCH)
                def _():
                    start_g(c + 1, (bg + 1) & 1)

                wait_g(c, bg & 1)
                extract(c, bg & 1, gsub, bg)

            start_w(g, gsub)
        return ()

    lax.fori_loop(0, NSG // 2, gpair, (), unroll=False)
    wait_w(NSG - 2, 0)
    wait_w(NSG - 1, 1)


def kernel(indices, table):
    idxT = indices.T.astype(jnp.int32)
    tab = table.reshape(250000, 128)

    mesh = plsc.VectorSubcoreMesh(core_axis_name="c", subcore_axis_name="s")
    k = functools.partial(
        pl.kernel,
        mesh=mesh,
        out_type=jax.ShapeDtypeStruct((B1, D, B0), jnp.float32),
        scratch_types=[
            pltpu.VMEM((56, BW), jnp.int32),
            pltpu.VMEM((BW * B1,), jnp.int32),
            pltpu.VMEM((BW * B1,), jnp.int32),
            pltpu.VMEM((2, CHUNK, 128), jnp.float32),
            pltpu.VMEM((2, SG, D, BW), jnp.float32),
            pltpu.SemaphoreType.DMA,
            pltpu.SemaphoreType.DMA,
        ],
        compiler_params=pltpu.CompilerParams(use_tc_tiling_on_sc=True,
                                             needs_layout_passes=False),
    )(_emb_kernel)
    out = k(idxT, tab)
    return jnp.transpose(out, (2, 0, 1))
